# baseline (speedup 1.0000x reference)
import functools

import jax
import jax.numpy as jnp
from jax import lax
from jax.experimental import pallas as pl
from jax.experimental.pallas import tpu as pltpu
from jax.experimental.pallas import tpu_sc as plsc

_NC = 2
_NS = 16
_NW = _NC * _NS

_BLOCK = 64


def _make_broadcast_kernel(S, D, dtype):
    b_per_w = S // _NW
    n_dma = b_per_w // _BLOCK
    mesh = plsc.VectorSubcoreMesh(core_axis_name="c", subcore_axis_name="s")

    @functools.partial(
        pl.kernel,
        out_type=jax.ShapeDtypeStruct((S, D), dtype),
        mesh=mesh,
        scratch_types=[
            pltpu.VMEM((_BLOCK,), jnp.int32),
            pltpu.VMEM((_BLOCK, D), dtype),
            pltpu.SemaphoreType.DMA,
            pltpu.SemaphoreType.DMA,
        ],
    )
    def broadcast_kernel(table_hbm, idx_hbm, out_hbm, idx_v, buf_v, gsem, wsem):
        wid = lax.axis_index("s") * _NC + lax.axis_index("c")
        base = wid * b_per_w
        pltpu.sync_copy(idx_hbm, idx_v)
        pltpu.async_copy(table_hbm.at[idx_v], buf_v, gsem).wait()
        copies = [
            pltpu.async_copy(
                buf_v, out_hbm.at[pl.ds(base + j * _BLOCK, _BLOCK)], wsem
            )
            for j in range(n_dma)
        ]
        for c in copies:
            c.wait()

    return broadcast_kernel


def kernel(embeddings, modality_embedding, token_type_id):
    S = embeddings.shape[1]
    D = modality_embedding.shape[1]
    idx = jnp.full((_BLOCK,), token_type_id, dtype=jnp.int32)
    fn = _make_broadcast_kernel(S, D, modality_embedding.dtype)
    return fn(modality_embedding, idx)


# speedup vs baseline: 1.7539x; 1.7539x over previous
import functools

import jax
import jax.numpy as jnp
from jax import lax
from jax.experimental import pallas as pl
from jax.experimental.pallas import tpu as pltpu
from jax.experimental.pallas import tpu_sc as plsc

_NC = 2
_NS = 16
_NW = _NC * _NS

_BLOCK = 16


def _make_broadcast_kernel(S, D, dtype):
    b_per_w = S // _NW
    n_dma = b_per_w // _BLOCK
    mesh = plsc.VectorSubcoreMesh(core_axis_name="c", subcore_axis_name="s")

    @functools.partial(
        pl.kernel,
        out_type=jax.ShapeDtypeStruct((S, D), dtype),
        mesh=mesh,
        scratch_types=[
            pltpu.VMEM((_BLOCK,), jnp.int32),
            pltpu.VMEM((_BLOCK, D), dtype),
            pltpu.SemaphoreType.DMA,
            pltpu.SemaphoreType.DMA,
        ],
    )
    def broadcast_kernel(table_hbm, idx_hbm, out_hbm, idx_v, buf_v, gsem, wsem):
        wid = lax.axis_index("s") * _NC + lax.axis_index("c")
        base = wid * b_per_w
        pltpu.sync_copy(idx_hbm, idx_v)
        pltpu.async_copy(table_hbm.at[idx_v], buf_v, gsem).wait()
        copies = [
            pltpu.async_copy(
                buf_v, out_hbm.at[pl.ds(base + j * _BLOCK, _BLOCK)], wsem
            )
            for j in range(n_dma)
        ]
        for c in copies:
            c.wait()

    return broadcast_kernel


def kernel(embeddings, modality_embedding, token_type_id):
    S = embeddings.shape[1]
    D = modality_embedding.shape[1]
    idx = jnp.full((_BLOCK,), token_type_id, dtype=jnp.int32)
    fn = _make_broadcast_kernel(S, D, modality_embedding.dtype)
    return fn(modality_embedding, idx)


# speedup vs baseline: 3.0854x; 1.7592x over previous
import functools

import jax
import jax.numpy as jnp
from jax import lax
from jax.experimental import pallas as pl
from jax.experimental.pallas import tpu as pltpu
from jax.experimental.pallas import tpu_sc as plsc

_NC = 2
_NS = 16
_NW = _NC * _NS

_BLOCK = 16


def _make_broadcast_kernel(S, D, dtype):
    b_per_w = S // _NW
    n_dma = b_per_w // _BLOCK
    mesh = plsc.VectorSubcoreMesh(core_axis_name="c", subcore_axis_name="s")

    @functools.partial(
        pl.kernel,
        out_type=jax.ShapeDtypeStruct((S, D), dtype),
        mesh=mesh,
        scratch_types=[
            pltpu.VMEM((_BLOCK,), jnp.int32),
            pltpu.VMEM((_BLOCK, D), dtype),
            pltpu.VMEM_SHARED((_BLOCK, D), dtype),
            pltpu.SemaphoreType.DMA,
            pltpu.SemaphoreType.DMA,
        ],
    )
    def broadcast_kernel(table_hbm, idx_hbm, out_hbm, idx_v, row_v, shared_v,
                         gsem, wsem):
        cid = lax.axis_index("c")
        sid = lax.axis_index("s")
        base = (cid * _NS + sid) * b_per_w

        @pl.when(sid == 0)
        def _():
            pltpu.sync_copy(idx_hbm, idx_v)
            pltpu.async_copy(table_hbm.at[idx_v], row_v, gsem).wait()
            pltpu.sync_copy(row_v, shared_v)

        plsc.subcore_barrier()

        copies = [
            pltpu.async_copy(
                shared_v, out_hbm.at[pl.ds(base + j * _BLOCK, _BLOCK)], wsem
            )
            for j in range(n_dma)
        ]
        for c in copies:
            c.wait()

    return broadcast_kernel


def kernel(embeddings, modality_embedding, token_type_id):
    S = embeddings.shape[1]
    D = modality_embedding.shape[1]
    idx = jnp.full((_BLOCK,), token_type_id, dtype=jnp.int32)
    fn = _make_broadcast_kernel(S, D, modality_embedding.dtype)
    return fn(modality_embedding, idx)
